# speedup vs baseline: 1.3637x; 1.3637x over previous
import jax
import jax.numpy as jnp
from jax.experimental import pallas as pl

_ROWS_PER_BLOCK = 2048


def _fused_body(word_ref, age_ref, w_ref, b_ref, tab_ref, out_ref):
    lin = jnp.dot(word_ref[...], w_ref[...],
                  preferred_element_type=jnp.float32) + b_ref[...]
    idx = age_ref[...]
    cols = jax.lax.broadcasted_iota(jnp.int32, (1, tab_ref.shape[0]), 1)
    onehot = (idx == cols).astype(jnp.float32)
    emb = jnp.dot(onehot, tab_ref[...], preferred_element_type=jnp.float32)
    out_ref[...] = jnp.concatenate((lin, emb), axis=-1)


def kernel(word, age, age_table, W, b):
    B, S, D = word.shape
    E = W.shape[1]
    A, EA = age_table.shape
    N = B * S

    word2 = word.reshape(N, D)
    age2 = jnp.asarray(age, jnp.int32).reshape(N, 1)
    AP = 128
    tab = jnp.zeros((AP, EA), jnp.float32).at[:A].set(age_table)
    b2 = b.reshape(1, E)

    R = _ROWS_PER_BLOCK
    grid = (N // R,)
    out = pl.pallas_call(
        _fused_body,
        grid=grid,
        in_specs=[
            pl.BlockSpec((R, D), lambda i: (i, 0)),
            pl.BlockSpec((R, 1), lambda i: (i, 0)),
            pl.BlockSpec((D, E), lambda i: (0, 0)),
            pl.BlockSpec((1, E), lambda i: (0, 0)),
            pl.BlockSpec((AP, EA), lambda i: (0, 0)),
        ],
        out_specs=pl.BlockSpec((R, E + EA), lambda i: (i, 0)),
        out_shape=jax.ShapeDtypeStruct((N, E + EA), jnp.float32),
    )(word2, age2, W, b2, tab)
    return out.reshape(B, S, E + EA)


# speedup vs baseline: 2.4238x; 1.7774x over previous
import jax
import jax.numpy as jnp
from jax.experimental import pallas as pl

_ROWS_PER_BLOCK = 256


def _fused_body(word_ref, age_ref, w_ref, b_ref, tab_ref, out_ref):
    R, S, D = word_ref.shape
    E = w_ref.shape[1]
    EA = tab_ref.shape[1]
    w2 = word_ref[...].reshape(R * S, D)
    lin = jnp.dot(w2, w_ref[...], preferred_element_type=jnp.float32) + b_ref[...]
    idx3 = age_ref[...][..., None]
    cols3 = jax.lax.broadcasted_iota(jnp.int32, (1, 1, tab_ref.shape[0]), 2)
    onehot = (idx3 == cols3).astype(jnp.float32).reshape(R * S, tab_ref.shape[0])
    emb = jnp.dot(onehot, tab_ref[...], preferred_element_type=jnp.float32)
    out = jnp.concatenate((lin, emb), axis=-1)
    out_ref[...] = out.reshape(R, S, E + EA)


def kernel(word, age, age_table, W, b):
    B, S, D = word.shape
    E = W.shape[1]
    A, EA = age_table.shape
    age32 = jnp.asarray(age, jnp.int32)
    AP = 128
    tab = jnp.zeros((AP, EA), jnp.float32).at[:A].set(age_table)
    b2 = b.reshape(1, E)

    R = _ROWS_PER_BLOCK
    grid = (B // R,)
    out = pl.pallas_call(
        _fused_body,
        grid=grid,
        in_specs=[
            pl.BlockSpec((R, S, D), lambda i: (i, 0, 0)),
            pl.BlockSpec((R, S), lambda i: (i, 0)),
            pl.BlockSpec((D, E), lambda i: (0, 0)),
            pl.BlockSpec((1, E), lambda i: (0, 0)),
            pl.BlockSpec((AP, EA), lambda i: (0, 0)),
        ],
        out_specs=pl.BlockSpec((R, S, E + EA), lambda i: (i, 0, 0)),
        out_shape=jax.ShapeDtypeStruct((B, S, E + EA), jnp.float32),
    )(word, age32, W, b2, tab)
    return out


# speedup vs baseline: 2.4480x; 1.0100x over previous
import jax
import jax.numpy as jnp
from jax.experimental import pallas as pl

_ROWS_PER_BLOCK = 512


def _fused_body(word_ref, age_ref, w_ref, b_ref, tab_ref, out_ref):
    R, S, D = word_ref.shape
    E = w_ref.shape[1]
    EA = tab_ref.shape[1]
    w2 = word_ref[...].reshape(R * S, D)
    lin = jnp.dot(w2, w_ref[...], preferred_element_type=jnp.float32) + b_ref[...]
    idx3 = age_ref[...][..., None]
    cols3 = jax.lax.broadcasted_iota(jnp.int32, (1, 1, tab_ref.shape[0]), 2)
    onehot = (idx3 == cols3).astype(jnp.float32).reshape(R * S, tab_ref.shape[0])
    emb = jnp.dot(onehot, tab_ref[...], preferred_element_type=jnp.float32)
    out = jnp.concatenate((lin, emb), axis=-1)
    out_ref[...] = out.reshape(R, S, E + EA)


def kernel(word, age, age_table, W, b):
    B, S, D = word.shape
    E = W.shape[1]
    A, EA = age_table.shape
    age32 = jnp.asarray(age, jnp.int32)
    AP = 128
    tab = jnp.zeros((AP, EA), jnp.float32).at[:A].set(age_table)
    b2 = b.reshape(1, E)

    R = _ROWS_PER_BLOCK
    grid = (B // R,)
    out = pl.pallas_call(
        _fused_body,
        grid=grid,
        in_specs=[
            pl.BlockSpec((R, S, D), lambda i: (i, 0, 0)),
            pl.BlockSpec((R, S), lambda i: (i, 0)),
            pl.BlockSpec((D, E), lambda i: (0, 0)),
            pl.BlockSpec((1, E), lambda i: (0, 0)),
            pl.BlockSpec((AP, EA), lambda i: (0, 0)),
        ],
        out_specs=pl.BlockSpec((R, S, E + EA), lambda i: (i, 0, 0)),
        out_shape=jax.ShapeDtypeStruct((B, S, E + EA), jnp.float32),
    )(word, age32, W, b2, tab)
    return out
